# Initial kernel scaffold; baseline (speedup 1.0000x reference)
import functools
import math

import jax
import jax.numpy as jnp
from jax import lax
from jax.experimental import pallas as pl
from jax.experimental.pallas import tpu as pltpu
from jax.experimental.pallas import tpu_sc as plsc

_IMG = 224
_P = 16
_C = 3
_V = 30522
_L = 77
_D = 1024
_E = 8
_H = 8
_NL = 2
_NCLS = 10
_DFF = 2048
_G = _IMG // _P
_NPATCH = _G * _G
_S = _NPATCH + _L
_B = 2
_SP = 288
_T = _B * _SP
_DH = _D // _H
_CPP = _C * _P * _P

_DFF_BLK = 1024
_NJ = _DFF // _DFF_BLK

_GELU_C = 0.7978845608028654

_NW = 32
_BPW = 8
_GROWS = _NW * _BPW


def _ln_f32(x, g, b):
    m = jnp.mean(x, axis=-1, keepdims=True)
    v = jnp.mean((x - m) * (x - m), axis=-1, keepdims=True)
    return (x - m) * lax.rsqrt(v + 1e-5) * g + b


def _gelu(x):
    x3 = x * x * x
    return 0.5 * x * (1.0 + jnp.tanh(_GELU_C * (x + 0.044715 * x3)))


def _sc_gather(table, idx):
    mesh = plsc.VectorSubcoreMesh(core_axis_name="c", subcore_axis_name="s")

    @functools.partial(
        pl.kernel,
        mesh=mesh,
        out_type=jax.ShapeDtypeStruct((_GROWS, _D), jnp.float32),
        scratch_types=[
            pltpu.VMEM((_BPW,), jnp.int32),
            pltpu.VMEM((_BPW, _D), jnp.float32),
            pltpu.SemaphoreType.DMA,
        ],
    )
    def k(table_hbm, idx_hbm, out_hbm, idx_v, rows_v, sem):
        wid = lax.axis_index("s") * 2 + lax.axis_index("c")
        base = wid * _BPW
        pltpu.sync_copy(idx_hbm.at[pl.ds(base, _BPW)], idx_v)
        pltpu.async_copy(table_hbm.at[idx_v], rows_v, sem).wait()
        pltpu.sync_copy(rows_v, out_hbm.at[pl.ds(base, _BPW)])

    return k(table, idx)


def _embed_body(pat_ref, wp_ref, bp_ref, pos_img_ref, pos_txt_ref,
                mod0_ref, mod1_ref, txt_ref, out_ref):
    wp = wp_ref[...]
    zpad = jnp.zeros((_SP - _S, _D), jnp.float32)
    parts = []
    for b in range(_B):
        img = jnp.dot(pat_ref[b * _NPATCH:(b + 1) * _NPATCH, :], wp,
                      preferred_element_type=jnp.float32)
        img = img + bp_ref[...] + pos_img_ref[...] + mod0_ref[...]
        txt = txt_ref[b * _L:(b + 1) * _L, :] + pos_txt_ref[...] + mod1_ref[...]
        parts += [img, txt, zpad]
    out_ref[...] = jnp.concatenate(parts, axis=0)


def _embed(patches, wp, bp, pos_img, pos_txt, mod0, mod1, txt):
    return pl.pallas_call(
        _embed_body,
        out_shape=jax.ShapeDtypeStruct((_T, _D), jnp.float32),
    )(patches, wp, bp, pos_img, pos_txt, mod0, mod1, txt)


def _attn_body(h_ref, mask_ref, g_ref, bn_ref, wqkv_ref, bqkv_ref,
               wo_ref, bo_ref, out_ref):
    x = h_ref[...]
    ln = _ln_f32(x, g_ref[...], bn_ref[...])
    qkv = jnp.dot(ln, wqkv_ref[...], preferred_element_type=jnp.float32)
    qkv = qkv + bqkv_ref[...]
    scale = 1.0 / math.sqrt(_DH)
    rows = []
    for b in range(_B):
        bias = (1.0 - mask_ref[b:b + 1, :]) * (-1e9)
        heads = []
        for hh in range(_H):
            q = qkv[b * _SP:(b + 1) * _SP, hh * _DH:(hh + 1) * _DH]
            k = qkv[b * _SP:(b + 1) * _SP, _D + hh * _DH:_D + (hh + 1) * _DH]
            v = qkv[b * _SP:(b + 1) * _SP, 2 * _D + hh * _DH:2 * _D + (hh + 1) * _DH]
            att = lax.dot_general(q, k, (((1,), (1,)), ((), ())),
                                  preferred_element_type=jnp.float32)
            att = att * scale + bias
            att = att - jnp.max(att, axis=-1, keepdims=True)
            pr = jnp.exp(att)
            pr = pr / jnp.sum(pr, axis=-1, keepdims=True)
            heads.append(jnp.dot(pr, v, preferred_element_type=jnp.float32))
        rows.append(jnp.concatenate(heads, axis=1))
    o = jnp.concatenate(rows, axis=0)
    out = jnp.dot(o, wo_ref[...], preferred_element_type=jnp.float32)
    out_ref[...] = x + out + bo_ref[...]


def _attn(h, mask, g, bn, wqkv, bqkv, wo, bo):
    return pl.pallas_call(
        _attn_body,
        out_shape=jax.ShapeDtypeStruct((_T, _D), jnp.float32),
    )(h, mask, g, bn, wqkv, bqkv, wo, bo)


def _moe_body(h_ref, g_ref, bn_ref, wr_ref, br_ref, w1_ref, b1_ref,
              w2_ref, b2_ref, out_ref, tln_ref, gates_ref):
    e = pl.program_id(0)
    j = pl.program_id(1)

    @pl.when((e == 0) & (j == 0))
    def _init():
        x = h_ref[...]
        ln = _ln_f32(x, g_ref[...], bn_ref[...])
        tln_ref[...] = ln
        logits = jnp.dot(ln, wr_ref[...], preferred_element_type=jnp.float32)
        logits = logits + br_ref[...]
        lanes = lax.broadcasted_iota(jnp.int32, (_T, _E), 1)
        m1 = jnp.max(logits, axis=-1, keepdims=True)
        a1 = jnp.min(jnp.where(logits == m1, lanes, _E), axis=-1, keepdims=True)
        l2 = jnp.where(lanes == a1, -jnp.inf, logits)
        m2 = jnp.max(l2, axis=-1, keepdims=True)
        a2 = jnp.min(jnp.where(l2 == m2, lanes, _E), axis=-1, keepdims=True)
        e2 = jnp.exp(m2 - m1)
        w1g = 1.0 / (1.0 + e2)
        w2g = e2 / (1.0 + e2)
        gates_ref[...] = (jnp.where(lanes == a1, w1g, 0.0)
                          + jnp.where(lanes == a2, w2g, 0.0))
        out_ref[...] = x

    tln = tln_ref[...]
    hmid = jnp.dot(tln, w1_ref[0], preferred_element_type=jnp.float32)
    hmid = _gelu(hmid + b1_ref[...])
    contrib = jnp.dot(hmid, w2_ref[0], preferred_element_type=jnp.float32)

    onehot = (lax.broadcasted_iota(jnp.int32, (_E, 1), 0) == e).astype(jnp.float32)
    gcol = jnp.dot(gates_ref[...], onehot, preferred_element_type=jnp.float32)

    @pl.when(j == 0)
    def _bias2():
        out_ref[...] += gcol * b2_ref[...]

    out_ref[...] += gcol * contrib


def _moe(h, g, bn, wr, br, w1, b1, w2, b2):
    return pl.pallas_call(
        _moe_body,
        grid=(_E, _NJ),
        in_specs=[
            pl.BlockSpec((_T, _D), lambda e, j: (0, 0)),
            pl.BlockSpec((1, _D), lambda e, j: (0, 0)),
            pl.BlockSpec((1, _D), lambda e, j: (0, 0)),
            pl.BlockSpec((_D, _E), lambda e, j: (0, 0)),
            pl.BlockSpec((1, _E), lambda e, j: (0, 0)),
            pl.BlockSpec((1, _D, _DFF_BLK), lambda e, j: (e, 0, j)),
            pl.BlockSpec((1, _DFF_BLK), lambda e, j: (e, j)),
            pl.BlockSpec((1, _DFF_BLK, _D), lambda e, j: (e, j, 0)),
            pl.BlockSpec((1, _D), lambda e, j: (e, 0)),
        ],
        out_specs=pl.BlockSpec((_T, _D), lambda e, j: (0, 0)),
        out_shape=jax.ShapeDtypeStruct((_T, _D), jnp.float32),
        scratch_shapes=[
            pltpu.VMEM((_T, _D), jnp.float32),
            pltpu.VMEM((_T, _E), jnp.float32),
        ],
    )(h, g, bn, wr, br, w1, b1, w2, b2)


def _head_body(h_ref, gf_ref, bf_ref, wc1_ref, bc1_ref, wc2_ref, bc2_ref,
               out_ref):
    ln = _ln_f32(h_ref[...], gf_ref[...], bf_ref[...])
    riota = lax.broadcasted_iota(jnp.int32, (_SP, 1), 0)
    w = jnp.where(riota < _S, 1.0 / _S, 0.0)
    fvs = []
    for b in range(_B):
        fvs.append(jnp.sum(ln[b * _SP:(b + 1) * _SP, :] * w, axis=0,
                           keepdims=True))
    fv = jnp.concatenate(fvs, axis=0)
    hcl = jnp.dot(fv, wc1_ref[...], preferred_element_type=jnp.float32)
    hcl = jnp.maximum(hcl + bc1_ref[...], 0.0)
    lg = jnp.dot(hcl, wc2_ref[...], preferred_element_type=jnp.float32)
    out_ref[...] = lg + bc2_ref[...]


def _head(h, gf, bf, wc1, bc1, wc2, bc2):
    return pl.pallas_call(
        _head_body,
        out_shape=jax.ShapeDtypeStruct((_B, _NCLS), jnp.float32),
    )(h, gf, bf, wc1, bc1, wc2, bc2)


def kernel(images, input_ids, attention_mask, params):
    p = params
    patches = images.reshape(_B, _C, _G, _P, _G, _P)
    patches = patches.transpose(0, 2, 4, 1, 3, 5).reshape(_B * _NPATCH, _CPP)

    ids = input_ids.reshape(-1).astype(jnp.int32)
    ids = jnp.concatenate([ids, jnp.zeros((_GROWS - _B * _L,), jnp.int32)])
    txt_rows = _sc_gather(p['tok_emb'], ids)[:_B * _L]

    h = _embed(patches, p['Wp'], p['bp'].reshape(1, _D), p['pos_img'],
               p['pos_txt'], p['mod'][0:1], p['mod'][1:2], txt_rows)

    mask = jnp.concatenate(
        [jnp.ones((_B, _NPATCH), jnp.float32),
         attention_mask.astype(jnp.float32),
         jnp.zeros((_B, _SP - _S), jnp.float32)], axis=1)

    for i in range(_NL):
        lp = p['layers'][i]
        h = _attn(h, mask, lp['g1'].reshape(1, _D), lp['b1n'].reshape(1, _D),
                  lp['Wqkv'], lp['bqkv'].reshape(1, 3 * _D), lp['Wo'],
                  lp['bo'].reshape(1, _D))
        h = _moe(h, lp['g2'].reshape(1, _D), lp['b2n'].reshape(1, _D),
                 lp['Wr'], lp['br'].reshape(1, _E), lp['W1'], lp['b1'],
                 lp['W2'], lp['b2'])

    return _head(h, p['gf'].reshape(1, _D), p['bf'].reshape(1, _D),
                 p['Wc1'], p['bc1'].reshape(1, _D // 2),
                 p['Wc2'], p['bc2'].reshape(1, _NCLS))


# speedup vs baseline: 1.5219x; 1.5219x over previous
import functools
import math

import jax
import jax.numpy as jnp
from jax import lax
from jax.experimental import pallas as pl
from jax.experimental.pallas import tpu as pltpu
from jax.experimental.pallas import tpu_sc as plsc

_IMG = 224
_P = 16
_C = 3
_V = 30522
_L = 77
_D = 1024
_E = 8
_H = 8
_NL = 2
_NCLS = 10
_DFF = 2048
_G = _IMG // _P
_NPATCH = _G * _G
_S = _NPATCH + _L
_B = 2
_SP = 288
_T = _B * _SP
_DH = _D // _H
_CPP = _C * _P * _P

_DFF_BLK = 1024
_NJ = _DFF // _DFF_BLK

_GELU_C = 0.7978845608028654

_NW = 32
_BPW = 8
_GROWS = _NW * _BPW


def _ln_f32(x, g, b):
    m = jnp.mean(x, axis=-1, keepdims=True)
    v = jnp.mean((x - m) * (x - m), axis=-1, keepdims=True)
    return (x - m) * lax.rsqrt(v + 1e-5) * g + b


def _gelu(x):
    x3 = x * x * x
    return 0.5 * x * (1.0 + jnp.tanh(_GELU_C * (x + 0.044715 * x3)))


def _sc_gather(table, idx):
    mesh = plsc.VectorSubcoreMesh(core_axis_name="c", subcore_axis_name="s")

    @functools.partial(
        pl.kernel,
        mesh=mesh,
        out_type=jax.ShapeDtypeStruct((_GROWS, _D), jnp.float32),
        scratch_types=[
            pltpu.VMEM((_BPW,), jnp.int32),
            pltpu.VMEM((_BPW, _D), jnp.float32),
            pltpu.SemaphoreType.DMA,
        ],
    )
    def k(table_hbm, idx_hbm, out_hbm, idx_v, rows_v, sem):
        wid = lax.axis_index("s") * 2 + lax.axis_index("c")
        base = wid * _BPW
        pltpu.sync_copy(idx_hbm.at[pl.ds(base, _BPW)], idx_v)
        pltpu.async_copy(table_hbm.at[idx_v], rows_v, sem).wait()
        pltpu.sync_copy(rows_v, out_hbm.at[pl.ds(base, _BPW)])

    return k(table, idx)


def _embed_body(pat_ref, wp_ref, bp_ref, pos_img_ref, pos_txt_ref,
                mod0_ref, mod1_ref, txt_ref, out_ref):
    wp = wp_ref[...]
    zpad = jnp.zeros((_SP - _S, _D), jnp.float32)
    parts = []
    for b in range(_B):
        img = jnp.dot(pat_ref[b * _NPATCH:(b + 1) * _NPATCH, :], wp,
                      preferred_element_type=jnp.float32)
        img = img + bp_ref[...] + pos_img_ref[...] + mod0_ref[...]
        txt = txt_ref[b * _L:(b + 1) * _L, :] + pos_txt_ref[...] + mod1_ref[...]
        parts += [img, txt, zpad]
    out_ref[...] = jnp.concatenate(parts, axis=0)


def _embed(patches, wp, bp, pos_img, pos_txt, mod0, mod1, txt):
    return pl.pallas_call(
        _embed_body,
        out_shape=jax.ShapeDtypeStruct((_T, _D), jnp.float32),
    )(patches, wp, bp, pos_img, pos_txt, mod0, mod1, txt)


def _attn_body(h_ref, mask_ref, g_ref, bn_ref, wqkv_ref, bqkv_ref,
               wo_ref, bo_ref, out_ref):
    x = h_ref[...]
    ln = _ln_f32(x, g_ref[...], bn_ref[...])
    qkv = jnp.dot(ln, wqkv_ref[...], preferred_element_type=jnp.float32)
    qkv = qkv + bqkv_ref[...]
    scale = 1.0 / math.sqrt(_DH)
    rows = []
    for b in range(_B):
        bias = (1.0 - mask_ref[b:b + 1, :]) * (-1e9)
        heads = []
        for hh in range(_H):
            q = qkv[b * _SP:(b + 1) * _SP, hh * _DH:(hh + 1) * _DH]
            k = qkv[b * _SP:(b + 1) * _SP, _D + hh * _DH:_D + (hh + 1) * _DH]
            v = qkv[b * _SP:(b + 1) * _SP, 2 * _D + hh * _DH:2 * _D + (hh + 1) * _DH]
            att = lax.dot_general(q, k, (((1,), (1,)), ((), ())),
                                  preferred_element_type=jnp.float32)
            att = att * scale + bias
            att = att - jnp.max(att, axis=-1, keepdims=True)
            pr = jnp.exp(att)
            pr = pr / jnp.sum(pr, axis=-1, keepdims=True)
            heads.append(jnp.dot(pr, v, preferred_element_type=jnp.float32))
        rows.append(jnp.concatenate(heads, axis=1))
    o = jnp.concatenate(rows, axis=0)
    out = jnp.dot(o, wo_ref[...], preferred_element_type=jnp.float32)
    out_ref[...] = x + out + bo_ref[...]


def _attn(h, mask, g, bn, wqkv, bqkv, wo, bo):
    return pl.pallas_call(
        _attn_body,
        out_shape=jax.ShapeDtypeStruct((_T, _D), jnp.float32),
    )(h, mask, g, bn, wqkv, bqkv, wo, bo)


def _moe_body(h_ref, g_ref, bn_ref, wr_ref, br_ref, w1_ref, b1_ref,
              w2_ref, b2_ref, out_ref, tln_ref, gates_ref):
    e = pl.program_id(0)
    j = pl.program_id(1)

    @pl.when((e == 0) & (j == 0))
    def _init():
        x = h_ref[...]
        ln = _ln_f32(x, g_ref[...], bn_ref[...])
        tln_ref[...] = ln
        logits = jnp.dot(ln, wr_ref[...], preferred_element_type=jnp.float32)
        logits = logits + br_ref[...]
        lanes = lax.broadcasted_iota(jnp.int32, (_T, _E), 1)
        m1 = jnp.max(logits, axis=-1, keepdims=True)
        a1 = jnp.min(jnp.where(logits == m1, lanes, _E), axis=-1, keepdims=True)
        l2 = jnp.where(lanes == a1, -jnp.inf, logits)
        m2 = jnp.max(l2, axis=-1, keepdims=True)
        a2 = jnp.min(jnp.where(l2 == m2, lanes, _E), axis=-1, keepdims=True)
        e2 = jnp.exp(m2 - m1)
        w1g = 1.0 / (1.0 + e2)
        w2g = e2 / (1.0 + e2)
        gates_ref[...] = (jnp.where(lanes == a1, w1g, 0.0)
                          + jnp.where(lanes == a2, w2g, 0.0))
        out_ref[...] = x

    tln = tln_ref[...]
    hmid = jnp.dot(tln, w1_ref[0], preferred_element_type=jnp.float32)
    hmid = _gelu(hmid + b1_ref[0])
    contrib = jnp.dot(hmid, w2_ref[0], preferred_element_type=jnp.float32)

    onehot = (lax.broadcasted_iota(jnp.int32, (_E, 1), 0) == e).astype(jnp.float32)
    gcol = jnp.dot(gates_ref[...], onehot, preferred_element_type=jnp.float32)

    @pl.when(j == 0)
    def _bias2():
        out_ref[...] += gcol * b2_ref[0]

    out_ref[...] += gcol * contrib


def _moe(h, g, bn, wr, br, w1, b1, w2, b2):
    return pl.pallas_call(
        _moe_body,
        grid=(_E, _NJ),
        in_specs=[
            pl.BlockSpec((_T, _D), lambda e, j: (0, 0)),
            pl.BlockSpec((1, _D), lambda e, j: (0, 0)),
            pl.BlockSpec((1, _D), lambda e, j: (0, 0)),
            pl.BlockSpec((_D, _E), lambda e, j: (0, 0)),
            pl.BlockSpec((1, _E), lambda e, j: (0, 0)),
            pl.BlockSpec((1, _D, _DFF_BLK), lambda e, j: (e, 0, j)),
            pl.BlockSpec((1, 1, _DFF_BLK), lambda e, j: (e * _NJ + j, 0, 0)),
            pl.BlockSpec((1, _DFF_BLK, _D), lambda e, j: (e, j, 0)),
            pl.BlockSpec((1, 1, _D), lambda e, j: (e, 0, 0)),
        ],
        out_specs=pl.BlockSpec((_T, _D), lambda e, j: (0, 0)),
        out_shape=jax.ShapeDtypeStruct((_T, _D), jnp.float32),
        scratch_shapes=[
            pltpu.VMEM((_T, _D), jnp.float32),
            pltpu.VMEM((_T, _E), jnp.float32),
        ],
    )(h, g, bn, wr, br, w1, b1.reshape(_E * _NJ, 1, _DFF_BLK),
      w2, b2.reshape(_E, 1, _D))


def _head_body(h_ref, gf_ref, bf_ref, wc1_ref, bc1_ref, wc2_ref, bc2_ref,
               out_ref):
    ln = _ln_f32(h_ref[...], gf_ref[...], bf_ref[...])
    riota = lax.broadcasted_iota(jnp.int32, (_SP, 1), 0)
    w = jnp.where(riota < _S, 1.0 / _S, 0.0)
    fvs = []
    for b in range(_B):
        fvs.append(jnp.sum(ln[b * _SP:(b + 1) * _SP, :] * w, axis=0,
                           keepdims=True))
    fv = jnp.concatenate(fvs, axis=0)
    hcl = jnp.dot(fv, wc1_ref[...], preferred_element_type=jnp.float32)
    hcl = jnp.maximum(hcl + bc1_ref[...], 0.0)
    lg = jnp.dot(hcl, wc2_ref[...], preferred_element_type=jnp.float32)
    out_ref[...] = lg + bc2_ref[...]


def _head(h, gf, bf, wc1, bc1, wc2, bc2):
    return pl.pallas_call(
        _head_body,
        out_shape=jax.ShapeDtypeStruct((_B, _NCLS), jnp.float32),
    )(h, gf, bf, wc1, bc1, wc2, bc2)


def kernel(images, input_ids, attention_mask, params):
    p = params
    patches = images.reshape(_B, _C, _G, _P, _G, _P)
    patches = patches.transpose(0, 2, 4, 1, 3, 5).reshape(_B * _NPATCH, _CPP)

    ids = input_ids.reshape(-1).astype(jnp.int32)
    ids = jnp.concatenate([ids, jnp.zeros((_GROWS - _B * _L,), jnp.int32)])
    txt_rows = _sc_gather(p['tok_emb'], ids)[:_B * _L]

    h = _embed(patches, p['Wp'], p['bp'].reshape(1, _D), p['pos_img'],
               p['pos_txt'], p['mod'][0:1], p['mod'][1:2], txt_rows)

    mask = jnp.concatenate(
        [jnp.ones((_B, _NPATCH), jnp.float32),
         attention_mask.astype(jnp.float32),
         jnp.zeros((_B, _SP - _S), jnp.float32)], axis=1)

    for i in range(_NL):
        lp = p['layers'][i]
        h = _attn(h, mask, lp['g1'].reshape(1, _D), lp['b1n'].reshape(1, _D),
                  lp['Wqkv'], lp['bqkv'].reshape(1, 3 * _D), lp['Wo'],
                  lp['bo'].reshape(1, _D))
        h = _moe(h, lp['g2'].reshape(1, _D), lp['b2n'].reshape(1, _D),
                 lp['Wr'], lp['br'].reshape(1, _E), lp['W1'], lp['b1'],
                 lp['W2'], lp['b2'])

    return _head(h, p['gf'].reshape(1, _D), p['bf'].reshape(1, _D),
                 p['Wc1'], p['bc1'].reshape(1, _D // 2),
                 p['Wc2'], p['bc2'].reshape(1, _NCLS))


# speedup vs baseline: 1.5506x; 1.0189x over previous
import functools
import math

import jax
import jax.numpy as jnp
from jax import lax
from jax.experimental import pallas as pl
from jax.experimental.pallas import tpu as pltpu
from jax.experimental.pallas import tpu_sc as plsc

_IMG = 224
_P = 16
_C = 3
_V = 30522
_L = 77
_D = 1024
_E = 8
_H = 8
_NL = 2
_NCLS = 10
_DFF = 2048
_G = _IMG // _P
_NPATCH = _G * _G
_S = _NPATCH + _L
_B = 2
_SP = 288
_T = _B * _SP
_DH = _D // _H
_CPP = _C * _P * _P

_DFF_BLK = 1024
_NJ = _DFF // _DFF_BLK

_GELU_C = 0.7978845608028654

_NW = 32
_BPW = 8
_GROWS = _NW * _BPW


def _ln_f32(x, g, b):
    m = jnp.mean(x, axis=-1, keepdims=True)
    v = jnp.mean((x - m) * (x - m), axis=-1, keepdims=True)
    return (x - m) * lax.rsqrt(v + 1e-5) * g + b


def _gelu(x):
    x3 = x * x * x
    return 0.5 * x * (1.0 + jnp.tanh(_GELU_C * (x + 0.044715 * x3)))


def _sc_gather(table, idx):
    mesh = plsc.VectorSubcoreMesh(core_axis_name="c", subcore_axis_name="s")

    @functools.partial(
        pl.kernel,
        mesh=mesh,
        out_type=jax.ShapeDtypeStruct((_GROWS, _D), jnp.float32),
        scratch_types=[
            pltpu.VMEM((_BPW,), jnp.int32),
            pltpu.VMEM((_BPW, _D), jnp.float32),
            pltpu.SemaphoreType.DMA,
        ],
    )
    def k(table_hbm, idx_hbm, out_hbm, idx_v, rows_v, sem):
        wid = lax.axis_index("s") * 2 + lax.axis_index("c")
        base = wid * _BPW
        pltpu.sync_copy(idx_hbm.at[pl.ds(base, _BPW)], idx_v)
        pltpu.async_copy(table_hbm.at[idx_v], rows_v, sem).wait()
        pltpu.sync_copy(rows_v, out_hbm.at[pl.ds(base, _BPW)])

    return k(table, idx)


def _embed_body(pat_ref, wp_ref, bp_ref, pos_img_ref, pos_txt_ref,
                mod0_ref, mod1_ref, txt_ref, out_ref):
    wp = wp_ref[...]
    zpad = jnp.zeros((_SP - _S, _D), jnp.float32)
    parts = []
    for b in range(_B):
        img = jnp.dot(pat_ref[b * _NPATCH:(b + 1) * _NPATCH, :], wp,
                      preferred_element_type=jnp.float32)
        img = img + bp_ref[...] + pos_img_ref[...] + mod0_ref[...]
        txt = txt_ref[b * _L:(b + 1) * _L, :] + pos_txt_ref[...] + mod1_ref[...]
        parts += [img, txt, zpad]
    out_ref[...] = jnp.concatenate(parts, axis=0)


def _embed(patches, wp, bp, pos_img, pos_txt, mod0, mod1, txt):
    return pl.pallas_call(
        _embed_body,
        out_shape=jax.ShapeDtypeStruct((_T, _D), jnp.float32),
    )(patches, wp, bp, pos_img, pos_txt, mod0, mod1, txt)


def _attn_body(h_ref, mask_ref, g_ref, bn_ref, wqkv_ref, bqkv_ref,
               wo_ref, bo_ref, out_ref):
    x = h_ref[...]
    ln = _ln_f32(x, g_ref[...], bn_ref[...])
    qkv = jnp.dot(ln.astype(jnp.bfloat16), wqkv_ref[...].astype(jnp.bfloat16),
                  preferred_element_type=jnp.float32)
    qkv = qkv + bqkv_ref[...]
    scale = 1.0 / math.sqrt(_DH)
    rows = []
    for b in range(_B):
        bias = (1.0 - mask_ref[b:b + 1, :]) * (-1e9)
        heads = []
        for hh in range(_H):
            q = qkv[b * _SP:(b + 1) * _SP, hh * _DH:(hh + 1) * _DH]
            k = qkv[b * _SP:(b + 1) * _SP, _D + hh * _DH:_D + (hh + 1) * _DH]
            v = qkv[b * _SP:(b + 1) * _SP, 2 * _D + hh * _DH:2 * _D + (hh + 1) * _DH]
            att = lax.dot_general(q.astype(jnp.bfloat16), k.astype(jnp.bfloat16),
                                  (((1,), (1,)), ((), ())),
                                  preferred_element_type=jnp.float32)
            att = att * scale + bias
            att = att - jnp.max(att, axis=-1, keepdims=True)
            pr = jnp.exp(att)
            pr = pr / jnp.sum(pr, axis=-1, keepdims=True)
            heads.append(jnp.dot(pr.astype(jnp.bfloat16),
                                 v.astype(jnp.bfloat16),
                                 preferred_element_type=jnp.float32))
        rows.append(jnp.concatenate(heads, axis=1))
    o = jnp.concatenate(rows, axis=0)
    out = jnp.dot(o.astype(jnp.bfloat16), wo_ref[...].astype(jnp.bfloat16),
                  preferred_element_type=jnp.float32)
    out_ref[...] = x + out + bo_ref[...]


def _attn(h, mask, g, bn, wqkv, bqkv, wo, bo):
    return pl.pallas_call(
        _attn_body,
        out_shape=jax.ShapeDtypeStruct((_T, _D), jnp.float32),
    )(h, mask, g, bn, wqkv, bqkv, wo, bo)


def _moe_body(h_ref, g_ref, bn_ref, wr_ref, br_ref, w1_ref, b1_ref,
              w2_ref, b2_ref, out_ref, tln_ref, tln16_ref, gates_ref):
    e = pl.program_id(0)
    j = pl.program_id(1)

    @pl.when((e == 0) & (j == 0))
    def _init():
        x = h_ref[...]
        ln = _ln_f32(x, g_ref[...], bn_ref[...])
        tln_ref[...] = ln
        tln16_ref[...] = ln.astype(jnp.bfloat16)
        logits = jnp.dot(ln, wr_ref[...], preferred_element_type=jnp.float32)
        logits = logits + br_ref[...]
        lanes = lax.broadcasted_iota(jnp.int32, (_T, _E), 1)
        m1 = jnp.max(logits, axis=-1, keepdims=True)
        a1 = jnp.min(jnp.where(logits == m1, lanes, _E), axis=-1, keepdims=True)
        l2 = jnp.where(lanes == a1, -jnp.inf, logits)
        m2 = jnp.max(l2, axis=-1, keepdims=True)
        a2 = jnp.min(jnp.where(l2 == m2, lanes, _E), axis=-1, keepdims=True)
        e2 = jnp.exp(m2 - m1)
        w1g = 1.0 / (1.0 + e2)
        w2g = e2 / (1.0 + e2)
        gates_ref[...] = (jnp.where(lanes == a1, w1g, 0.0)
                          + jnp.where(lanes == a2, w2g, 0.0))
        out_ref[...] = x

    tln = tln16_ref[...]
    hmid = jnp.dot(tln, w1_ref[0].astype(jnp.bfloat16),
                   preferred_element_type=jnp.float32)
    hmid = _gelu(hmid + b1_ref[0])
    contrib = jnp.dot(hmid.astype(jnp.bfloat16),
                      w2_ref[0].astype(jnp.bfloat16),
                      preferred_element_type=jnp.float32)

    onehot = (lax.broadcasted_iota(jnp.int32, (_E, 1), 0) == e).astype(jnp.float32)
    gcol = jnp.dot(gates_ref[...], onehot, preferred_element_type=jnp.float32)

    @pl.when(j == 0)
    def _bias2():
        out_ref[...] += gcol * b2_ref[0]

    out_ref[...] += gcol * contrib


def _moe(h, g, bn, wr, br, w1, b1, w2, b2):
    return pl.pallas_call(
        _moe_body,
        grid=(_E, _NJ),
        in_specs=[
            pl.BlockSpec((_T, _D), lambda e, j: (0, 0)),
            pl.BlockSpec((1, _D), lambda e, j: (0, 0)),
            pl.BlockSpec((1, _D), lambda e, j: (0, 0)),
            pl.BlockSpec((_D, _E), lambda e, j: (0, 0)),
            pl.BlockSpec((1, _E), lambda e, j: (0, 0)),
            pl.BlockSpec((1, _D, _DFF_BLK), lambda e, j: (e, 0, j)),
            pl.BlockSpec((1, 1, _DFF_BLK), lambda e, j: (e * _NJ + j, 0, 0)),
            pl.BlockSpec((1, _DFF_BLK, _D), lambda e, j: (e, j, 0)),
            pl.BlockSpec((1, 1, _D), lambda e, j: (e, 0, 0)),
        ],
        out_specs=pl.BlockSpec((_T, _D), lambda e, j: (0, 0)),
        out_shape=jax.ShapeDtypeStruct((_T, _D), jnp.float32),
        scratch_shapes=[
            pltpu.VMEM((_T, _D), jnp.float32),
            pltpu.VMEM((_T, _D), jnp.bfloat16),
            pltpu.VMEM((_T, _E), jnp.float32),
        ],
    )(h, g, bn, wr, br, w1, b1.reshape(_E * _NJ, 1, _DFF_BLK),
      w2, b2.reshape(_E, 1, _D))


def _head_body(h_ref, gf_ref, bf_ref, wc1_ref, bc1_ref, wc2_ref, bc2_ref,
               out_ref):
    ln = _ln_f32(h_ref[...], gf_ref[...], bf_ref[...])
    riota = lax.broadcasted_iota(jnp.int32, (_SP, 1), 0)
    w = jnp.where(riota < _S, 1.0 / _S, 0.0)
    fvs = []
    for b in range(_B):
        fvs.append(jnp.sum(ln[b * _SP:(b + 1) * _SP, :] * w, axis=0,
                           keepdims=True))
    fv = jnp.concatenate(fvs, axis=0)
    hcl = jnp.dot(fv, wc1_ref[...], preferred_element_type=jnp.float32)
    hcl = jnp.maximum(hcl + bc1_ref[...], 0.0)
    lg = jnp.dot(hcl, wc2_ref[...], preferred_element_type=jnp.float32)
    out_ref[...] = lg + bc2_ref[...]


def _head(h, gf, bf, wc1, bc1, wc2, bc2):
    return pl.pallas_call(
        _head_body,
        out_shape=jax.ShapeDtypeStruct((_B, _NCLS), jnp.float32),
    )(h, gf, bf, wc1, bc1, wc2, bc2)


def kernel(images, input_ids, attention_mask, params):
    p = params
    patches = images.reshape(_B, _C, _G, _P, _G, _P)
    patches = patches.transpose(0, 2, 4, 1, 3, 5).reshape(_B * _NPATCH, _CPP)

    ids = input_ids.reshape(-1).astype(jnp.int32)
    ids = jnp.concatenate([ids, jnp.zeros((_GROWS - _B * _L,), jnp.int32)])
    txt_rows = _sc_gather(p['tok_emb'], ids)[:_B * _L]

    h = _embed(patches, p['Wp'], p['bp'].reshape(1, _D), p['pos_img'],
               p['pos_txt'], p['mod'][0:1], p['mod'][1:2], txt_rows)

    mask = jnp.concatenate(
        [jnp.ones((_B, _NPATCH), jnp.float32),
         attention_mask.astype(jnp.float32),
         jnp.zeros((_B, _SP - _S), jnp.float32)], axis=1)

    for i in range(_NL):
        lp = p['layers'][i]
        h = _attn(h, mask, lp['g1'].reshape(1, _D), lp['b1n'].reshape(1, _D),
                  lp['Wqkv'], lp['bqkv'].reshape(1, 3 * _D), lp['Wo'],
                  lp['bo'].reshape(1, _D))
        h = _moe(h, lp['g2'].reshape(1, _D), lp['b2n'].reshape(1, _D),
                 lp['Wr'], lp['br'].reshape(1, _E), lp['W1'], lp['b1'],
                 lp['W2'], lp['b2'])

    return _head(h, p['gf'].reshape(1, _D), p['bf'].reshape(1, _D),
                 p['Wc1'], p['bc1'].reshape(1, _D // 2),
                 p['Wc2'], p['bc2'].reshape(1, _NCLS))


# speedup vs baseline: 1.6056x; 1.0355x over previous
import functools
import math

import jax
import jax.numpy as jnp
from jax import lax
from jax.experimental import pallas as pl
from jax.experimental.pallas import tpu as pltpu
from jax.experimental.pallas import tpu_sc as plsc

_IMG = 224
_P = 16
_C = 3
_V = 30522
_L = 77
_D = 1024
_E = 8
_H = 8
_NL = 2
_NCLS = 10
_DFF = 2048
_G = _IMG // _P
_NPATCH = _G * _G
_S = _NPATCH + _L
_B = 2
_SP = 288
_T = _B * _SP
_DH = _D // _H
_CPP = _C * _P * _P

_DFF_BLK = 1024
_NJ = _DFF // _DFF_BLK

_GELU_C = 0.7978845608028654

_NW = 32
_BPW = 8
_GROWS = _NW * _BPW


def _ln_f32(x, g, b):
    m = jnp.mean(x, axis=-1, keepdims=True)
    v = jnp.mean((x - m) * (x - m), axis=-1, keepdims=True)
    return (x - m) * lax.rsqrt(v + 1e-5) * g + b


def _gelu(x):
    x3 = x * x * x
    return 0.5 * x * (1.0 + jnp.tanh(_GELU_C * (x + 0.044715 * x3)))


def _sc_gather(table, idx):
    mesh = plsc.VectorSubcoreMesh(core_axis_name="c", subcore_axis_name="s")

    @functools.partial(
        pl.kernel,
        mesh=mesh,
        out_type=jax.ShapeDtypeStruct((_GROWS, _D), jnp.float32),
        scratch_types=[
            pltpu.VMEM((_BPW,), jnp.int32),
            pltpu.VMEM((_BPW, _D), jnp.float32),
            pltpu.SemaphoreType.DMA,
        ],
    )
    def k(table_hbm, idx_hbm, out_hbm, idx_v, rows_v, sem):
        wid = lax.axis_index("s") * 2 + lax.axis_index("c")
        base = wid * _BPW
        pltpu.sync_copy(idx_hbm.at[pl.ds(base, _BPW)], idx_v)
        pltpu.async_copy(table_hbm.at[idx_v], rows_v, sem).wait()
        pltpu.sync_copy(rows_v, out_hbm.at[pl.ds(base, _BPW)])

    return k(table, idx)


def _embed_body(pat_ref, wp_ref, bp_ref, pos_img_ref, pos_txt_ref,
                mod0_ref, mod1_ref, txt_ref, out_ref):
    wp = wp_ref[...]
    zpad = jnp.zeros((_SP - _S, _D), jnp.float32)
    parts = []
    for b in range(_B):
        img = jnp.dot(pat_ref[b * _NPATCH:(b + 1) * _NPATCH, :], wp,
                      preferred_element_type=jnp.float32)
        img = img + bp_ref[...] + pos_img_ref[...] + mod0_ref[...]
        txt = txt_ref[b * _L:(b + 1) * _L, :] + pos_txt_ref[...] + mod1_ref[...]
        parts += [img, txt, zpad]
    out_ref[...] = jnp.concatenate(parts, axis=0)


def _embed(patches, wp, bp, pos_img, pos_txt, mod0, mod1, txt):
    return pl.pallas_call(
        _embed_body,
        out_shape=jax.ShapeDtypeStruct((_T, _D), jnp.float32),
    )(patches, wp, bp, pos_img, pos_txt, mod0, mod1, txt)


def _attn_body(h_ref, mask_ref, g_ref, bn_ref, wqkv_ref, bqkv_ref,
               wo_ref, bo_ref, out_ref):
    x = h_ref[...]
    ln = _ln_f32(x, g_ref[...], bn_ref[...])
    qkv = jnp.dot(ln.astype(jnp.bfloat16), wqkv_ref[...].astype(jnp.bfloat16),
                  preferred_element_type=jnp.float32)
    qkv = qkv + bqkv_ref[...]
    scale = 1.0 / math.sqrt(_DH)
    rows = []
    for b in range(_B):
        bias = (1.0 - mask_ref[b:b + 1, :]) * (-1e9)
        heads = []
        for hh in range(_H):
            q = qkv[b * _SP:(b + 1) * _SP, hh * _DH:(hh + 1) * _DH]
            k = qkv[b * _SP:(b + 1) * _SP, _D + hh * _DH:_D + (hh + 1) * _DH]
            v = qkv[b * _SP:(b + 1) * _SP, 2 * _D + hh * _DH:2 * _D + (hh + 1) * _DH]
            att = lax.dot_general(q.astype(jnp.bfloat16), k.astype(jnp.bfloat16),
                                  (((1,), (1,)), ((), ())),
                                  preferred_element_type=jnp.float32)
            att = att * scale + bias
            att = att - jnp.max(att, axis=-1, keepdims=True)
            pr = jnp.exp(att)
            pr = pr / jnp.sum(pr, axis=-1, keepdims=True)
            heads.append(jnp.dot(pr.astype(jnp.bfloat16),
                                 v.astype(jnp.bfloat16),
                                 preferred_element_type=jnp.float32))
        rows.append(jnp.concatenate(heads, axis=1))
    o = jnp.concatenate(rows, axis=0)
    out = jnp.dot(o.astype(jnp.bfloat16), wo_ref[...].astype(jnp.bfloat16),
                  preferred_element_type=jnp.float32)
    out_ref[...] = x + out + bo_ref[...]


def _attn(h, mask, g, bn, wqkv, bqkv, wo, bo):
    return pl.pallas_call(
        _attn_body,
        out_shape=jax.ShapeDtypeStruct((_T, _D), jnp.float32),
    )(h, mask, g, bn, wqkv, bqkv, wo, bo)


def _moe_body(h_ref, g_ref, bn_ref, wr_ref, br_ref, w1_ref, b1_ref,
              w2_ref, b2_ref, out_ref, tln_ref, tln16_ref, gates_ref):
    e = pl.program_id(0)

    @pl.when(e == 0)
    def _init():
        x = h_ref[...]
        ln = _ln_f32(x, g_ref[...], bn_ref[...])
        tln_ref[...] = ln
        tln16_ref[...] = ln.astype(jnp.bfloat16)
        logits = jnp.dot(ln, wr_ref[...], preferred_element_type=jnp.float32)
        logits = logits + br_ref[...]
        lanes = lax.broadcasted_iota(jnp.int32, (_T, _E), 1)
        m1 = jnp.max(logits, axis=-1, keepdims=True)
        a1 = jnp.min(jnp.where(logits == m1, lanes, _E), axis=-1, keepdims=True)
        l2 = jnp.where(lanes == a1, -jnp.inf, logits)
        m2 = jnp.max(l2, axis=-1, keepdims=True)
        a2 = jnp.min(jnp.where(l2 == m2, lanes, _E), axis=-1, keepdims=True)
        e2 = jnp.exp(m2 - m1)
        w1g = 1.0 / (1.0 + e2)
        w2g = e2 / (1.0 + e2)
        gates_ref[...] = (jnp.where(lanes == a1, w1g, 0.0)
                          + jnp.where(lanes == a2, w2g, 0.0))
        out_ref[...] = x

    tln = tln16_ref[...]
    hmid = jnp.dot(tln, w1_ref[0].astype(jnp.bfloat16),
                   preferred_element_type=jnp.float32)
    hmid = _gelu(hmid + b1_ref[0])
    contrib = jnp.dot(hmid.astype(jnp.bfloat16),
                      w2_ref[0].astype(jnp.bfloat16),
                      preferred_element_type=jnp.float32)

    onehot = (lax.broadcasted_iota(jnp.int32, (_E, 1), 0) == e).astype(jnp.float32)
    gcol = jnp.dot(gates_ref[...], onehot, preferred_element_type=jnp.float32)

    out_ref[...] += gcol * (contrib + b2_ref[0])


def _moe(h, g, bn, wr, br, w1, b1, w2, b2):
    return pl.pallas_call(
        _moe_body,
        grid=(_E,),
        in_specs=[
            pl.BlockSpec((_T, _D), lambda e: (0, 0)),
            pl.BlockSpec((1, _D), lambda e: (0, 0)),
            pl.BlockSpec((1, _D), lambda e: (0, 0)),
            pl.BlockSpec((_D, _E), lambda e: (0, 0)),
            pl.BlockSpec((1, _E), lambda e: (0, 0)),
            pl.BlockSpec((1, _D, _DFF), lambda e: (e, 0, 0)),
            pl.BlockSpec((1, 1, _DFF), lambda e: (e, 0, 0)),
            pl.BlockSpec((1, _DFF, _D), lambda e: (e, 0, 0)),
            pl.BlockSpec((1, 1, _D), lambda e: (e, 0, 0)),
        ],
        out_specs=pl.BlockSpec((_T, _D), lambda e: (0, 0)),
        out_shape=jax.ShapeDtypeStruct((_T, _D), jnp.float32),
        scratch_shapes=[
            pltpu.VMEM((_T, _D), jnp.float32),
            pltpu.VMEM((_T, _D), jnp.bfloat16),
            pltpu.VMEM((_T, _E), jnp.float32),
        ],
    )(h, g, bn, wr, br, w1, b1.reshape(_E, 1, _DFF),
      w2, b2.reshape(_E, 1, _D))


def _head_body(h_ref, gf_ref, bf_ref, wc1_ref, bc1_ref, wc2_ref, bc2_ref,
               out_ref):
    ln = _ln_f32(h_ref[...], gf_ref[...], bf_ref[...])
    riota = lax.broadcasted_iota(jnp.int32, (_SP, 1), 0)
    w = jnp.where(riota < _S, 1.0 / _S, 0.0)
    fvs = []
    for b in range(_B):
        fvs.append(jnp.sum(ln[b * _SP:(b + 1) * _SP, :] * w, axis=0,
                           keepdims=True))
    fv = jnp.concatenate(fvs, axis=0)
    hcl = jnp.dot(fv, wc1_ref[...], preferred_element_type=jnp.float32)
    hcl = jnp.maximum(hcl + bc1_ref[...], 0.0)
    lg = jnp.dot(hcl, wc2_ref[...], preferred_element_type=jnp.float32)
    out_ref[...] = lg + bc2_ref[...]


def _head(h, gf, bf, wc1, bc1, wc2, bc2):
    return pl.pallas_call(
        _head_body,
        out_shape=jax.ShapeDtypeStruct((_B, _NCLS), jnp.float32),
    )(h, gf, bf, wc1, bc1, wc2, bc2)


def kernel(images, input_ids, attention_mask, params):
    p = params
    patches = images.reshape(_B, _C, _G, _P, _G, _P)
    patches = patches.transpose(0, 2, 4, 1, 3, 5).reshape(_B * _NPATCH, _CPP)

    ids = input_ids.reshape(-1).astype(jnp.int32)
    ids = jnp.concatenate([ids, jnp.zeros((_GROWS - _B * _L,), jnp.int32)])
    txt_rows = _sc_gather(p['tok_emb'], ids)[:_B * _L]

    h = _embed(patches, p['Wp'], p['bp'].reshape(1, _D), p['pos_img'],
               p['pos_txt'], p['mod'][0:1], p['mod'][1:2], txt_rows)

    mask = jnp.concatenate(
        [jnp.ones((_B, _NPATCH), jnp.float32),
         attention_mask.astype(jnp.float32),
         jnp.zeros((_B, _SP - _S), jnp.float32)], axis=1)

    for i in range(_NL):
        lp = p['layers'][i]
        h = _attn(h, mask, lp['g1'].reshape(1, _D), lp['b1n'].reshape(1, _D),
                  lp['Wqkv'], lp['bqkv'].reshape(1, 3 * _D), lp['Wo'],
                  lp['bo'].reshape(1, _D))
        h = _moe(h, lp['g2'].reshape(1, _D), lp['b2n'].reshape(1, _D),
                 lp['Wr'], lp['br'].reshape(1, _E), lp['W1'], lp['b1'],
                 lp['W2'], lp['b2'])

    return _head(h, p['gf'].reshape(1, _D), p['bf'].reshape(1, _D),
                 p['Wc1'], p['bc1'].reshape(1, _D // 2),
                 p['Wc2'], p['bc2'].reshape(1, _NCLS))
